# Initial kernel scaffold; baseline (speedup 1.0000x reference)
import jax
import jax.numpy as jnp
from jax.experimental import pallas as pl


def kernel(x, y, neg, WI, WO, seq_embed, fc1_w, fc1_b, fc2_w, fc2_b):
    raise NotImplementedError("write your pallas kernel here")


# speedup vs baseline: 2.8787x; 2.8787x over previous
import functools

import jax
import jax.numpy as jnp
from jax import lax
from jax.experimental import pallas as pl
from jax.experimental.pallas import tpu as pltpu
from jax.experimental.pallas import tpu_sc as plsc

B = 4096
E = 128
SD = 128
NEG = 5

NC = 2
NS = 16
NW = NC * NS
BPW = B // NW
NPW = BPW * NEG

@functools.cache
def _build_sc_gather():
    mesh = plsc.VectorSubcoreMesh(core_axis_name="c", subcore_axis_name="s")

    @functools.partial(
        pl.kernel,
        mesh=mesh,
        out_type=(
            jax.ShapeDtypeStruct((B, E), jnp.float32),
            jax.ShapeDtypeStruct((B, E), jnp.float32),
            jax.ShapeDtypeStruct((B, SD), jnp.float32),
            jax.ShapeDtypeStruct((B * NEG, E), jnp.float32),
            jax.ShapeDtypeStruct((B * NEG, SD), jnp.float32),
        ),
        scratch_types=[
            pltpu.VMEM((BPW,), jnp.int32),
            pltpu.VMEM((BPW, E), jnp.float32),
            pltpu.SemaphoreType.DMA,
        ],
    )
    def _sc_gather(x_h, y_h, negf_h, WI_h, WO_h, SE_h,
                   vI_h, WOy_h, SEy_h, WOn_h, SEn_h,
                   idx_v, rows_v, sem):
        wid = lax.axis_index("s") * NC + lax.axis_index("c")
        base = wid * BPW
        nbase = wid * NPW

        def do(idx_h, ioff, tab_h, out_h, ooff):
            pltpu.sync_copy(idx_h.at[pl.ds(ioff, BPW)], idx_v)
            pltpu.async_copy(tab_h.at[idx_v], rows_v, sem).wait()
            pltpu.sync_copy(rows_v, out_h.at[pl.ds(ooff, BPW)])

        do(x_h, base, WI_h, vI_h, base)
        do(y_h, base, WO_h, WOy_h, base)
        do(y_h, base, SE_h, SEy_h, base)
        for j in range(NEG):
            do(negf_h, nbase + j * BPW, WO_h, WOn_h, nbase + j * BPW)
            do(negf_h, nbase + j * BPW, SE_h, SEn_h, nbase + j * BPW)

    return _sc_gather


BC = 512


def _log_sigmoid(z):
    return jnp.minimum(z, 0.0) - jnp.log1p(jnp.exp(-jnp.abs(z)))


def _tc_body(vI_r, WOy_r, SEy_r, WOn_r, SEn_r, f1w_r, f2w_r, f1b_r, f2b_r,
             out_r):
    i = pl.program_id(0)
    vI = vI_r[...]
    U = jnp.dot(vI, f2w_r[...], preferred_element_type=jnp.float32)
    U1 = U[:, :E]
    U2 = U[:, E:]
    f1w = f1w_r[...]
    A1 = jnp.dot(U1, f1w, preferred_element_type=jnp.float32)
    A2 = jnp.dot(U2, f1w, preferred_element_type=jnp.float32)
    f1b = f1b_r[...]
    f2b = f2b_r[...]
    cI = jnp.sum(vI * f2b, axis=1)
    c1 = jnp.sum(U1 * f1b, axis=1)
    c2 = jnp.sum(U2 * f1b, axis=1)
    pos = jnp.sum(U1 * WOy_r[...] + A2 * SEy_r[...], axis=1) + c2 + cI
    WOn = WOn_r[...].reshape(BC, NEG, E)
    SEn = SEn_r[...].reshape(BC, NEG, E)
    negraw = jnp.sum(A1[:, None, :] * WOn + U2[:, None, :] * SEn, axis=2)
    negraw = negraw + (c1 + cI)[:, None]
    part = (-jnp.sum(_log_sigmoid(pos)) / B
            - jnp.sum(_log_sigmoid(-negraw)))

    @pl.when(i == 0)
    def _init():
        out_r[0, 0] = part

    @pl.when(i > 0)
    def _acc():
        out_r[0, 0] = out_r[0, 0] + part


def _tc_compute(vI, WOy, SEy, WOn, SEn, f1w, f2w, f1b, f2b):
    grid = (B // BC,)
    return pl.pallas_call(
        _tc_body,
        grid=grid,
        in_specs=[
            pl.BlockSpec((BC, E), lambda i: (i, 0)),
            pl.BlockSpec((BC, E), lambda i: (i, 0)),
            pl.BlockSpec((BC, SD), lambda i: (i, 0)),
            pl.BlockSpec((BC * NEG, E), lambda i: (i, 0)),
            pl.BlockSpec((BC * NEG, SD), lambda i: (i, 0)),
            pl.BlockSpec((SD, SD), lambda i: (0, 0)),
            pl.BlockSpec((E, E + SD), lambda i: (0, 0)),
            pl.BlockSpec((1, SD), lambda i: (0, 0)),
            pl.BlockSpec((1, E), lambda i: (0, 0)),
        ],
        out_specs=pl.BlockSpec((1, 1), lambda i: (0, 0),
                               memory_space=pltpu.SMEM),
        out_shape=jax.ShapeDtypeStruct((1, 1), jnp.float32),
    )(vI, WOy, SEy, WOn, SEn, f1w, f2w, f1b, f2b)


def kernel(x, y, neg, WI, WO, seq_embed, fc1_w, fc1_b, fc2_w, fc2_b):
    xi = x.astype(jnp.int32)
    yi = y.astype(jnp.int32)
    negf = neg.astype(jnp.int32).reshape(B * NEG)
    vI, WOy, SEy, WOn, SEn = _build_sc_gather()(xi, yi, negf, WI, WO,
                                                seq_embed)
    out = _tc_compute(vI, WOy, SEy, WOn, SEn,
                      fc1_w, fc2_w,
                      fc1_b.reshape(1, SD), fc2_b.reshape(1, E))
    return out[0, 0]


# speedup vs baseline: 4.6937x; 1.6305x over previous
import functools

import jax
import jax.numpy as jnp
from jax import lax
from jax.experimental import pallas as pl
from jax.experimental.pallas import tpu as pltpu
from jax.experimental.pallas import tpu_sc as plsc

B = 4096
E = 128
SD = 128
NEG = 5

NC = 2
NS = 16
NW = NC * NS
BPW = B // NW


@functools.cache
def _build_sc_gather():
    mesh = plsc.VectorSubcoreMesh(core_axis_name="c", subcore_axis_name="s")

    @functools.partial(
        pl.kernel,
        mesh=mesh,
        out_type=(
            jax.ShapeDtypeStruct((B, E), jnp.float32),
            jax.ShapeDtypeStruct((B, E), jnp.float32),
            jax.ShapeDtypeStruct((B, SD), jnp.float32),
            jax.ShapeDtypeStruct((NEG * B, E), jnp.float32),
            jax.ShapeDtypeStruct((NEG * B, SD), jnp.float32),
        ),
        scratch_types=[
            pltpu.VMEM((7, BPW), jnp.int32),
            pltpu.VMEM((BPW, E), jnp.float32),
            pltpu.VMEM((BPW, E), jnp.float32),
            pltpu.SemaphoreType.DMA,
            pltpu.SemaphoreType.DMA,
        ],
    )
    def _sc_gather(x_h, y_h, negt_h, WI_h, WO_h, SE_h,
                   vI_h, WOy_h, SEy_h, WOn_h, SEn_h,
                   idx_v, buf0, buf1, sem0, sem1):
        wid = lax.axis_index("s") * NC + lax.axis_index("c")
        base = wid * BPW

        pltpu.sync_copy(x_h.at[pl.ds(base, BPW)], idx_v.at[0])
        pltpu.sync_copy(y_h.at[pl.ds(base, BPW)], idx_v.at[1])
        for n in range(NEG):
            pltpu.sync_copy(negt_h.at[pl.ds(n * B + base, BPW)],
                            idx_v.at[2 + n])

        tasks = [(0, WI_h, vI_h, base), (1, WO_h, WOy_h, base),
                 (1, SE_h, SEy_h, base)]
        for n in range(NEG):
            tasks.append((2 + n, WO_h, WOn_h, n * B + base))
        for n in range(NEG):
            tasks.append((2 + n, SE_h, SEn_h, n * B + base))

        bufs = (buf0, buf1)
        sems = (sem0, sem1)
        copies = [None, None]

        def start(t, slot):
            j, tab_h, _, _ = tasks[t]
            copies[slot] = pltpu.async_copy(tab_h.at[idx_v.at[j]],
                                            bufs[slot], sems[slot])

        start(0, 0)
        for t in range(len(tasks)):
            slot = t % 2
            if t + 1 < len(tasks):
                start(t + 1, 1 - slot)
            copies[slot].wait()
            _, _, out_h, ooff = tasks[t]
            pltpu.sync_copy(bufs[slot], out_h.at[pl.ds(ooff, BPW)])

    return _sc_gather


BC = 512


def _log_sigmoid(z):
    return jnp.minimum(z, 0.0) - jnp.log1p(jnp.exp(-jnp.abs(z)))


def _tc_body(vI_r, WOy_r, SEy_r, WOn_r, SEn_r, f1w_r, f2w_r, f1b_r, f2b_r,
             out_r):
    i = pl.program_id(0)
    vI = vI_r[...]
    U = jnp.dot(vI, f2w_r[...], preferred_element_type=jnp.float32)
    U1 = U[:, :E]
    U2 = U[:, E:]
    f1w = f1w_r[...]
    A1 = jnp.dot(U1, f1w, preferred_element_type=jnp.float32)
    A2 = jnp.dot(U2, f1w, preferred_element_type=jnp.float32)
    f1b = f1b_r[...]
    f2b = f2b_r[...]
    cI = jnp.sum(vI * f2b, axis=1)
    c1 = jnp.sum(U1 * f1b, axis=1)
    c2 = jnp.sum(U2 * f1b, axis=1)
    pos = jnp.sum(U1 * WOy_r[...] + A2 * SEy_r[...], axis=1) + c2 + cI
    part = -jnp.sum(_log_sigmoid(pos)) / B
    cneg = c1 + cI
    for n in range(NEG):
        zn = jnp.sum(A1 * WOn_r[n] + U2 * SEn_r[n], axis=1) + cneg
        part = part - jnp.sum(_log_sigmoid(-zn))

    @pl.when(i == 0)
    def _init():
        out_r[0, 0] = part

    @pl.when(i > 0)
    def _acc():
        out_r[0, 0] = out_r[0, 0] + part


def _tc_compute(vI, WOy, SEy, WOn, SEn, f1w, f2w, f1b, f2b):
    grid = (B // BC,)
    return pl.pallas_call(
        _tc_body,
        grid=grid,
        in_specs=[
            pl.BlockSpec((BC, E), lambda i: (i, 0)),
            pl.BlockSpec((BC, E), lambda i: (i, 0)),
            pl.BlockSpec((BC, SD), lambda i: (i, 0)),
            pl.BlockSpec((NEG, BC, E), lambda i: (0, i, 0)),
            pl.BlockSpec((NEG, BC, SD), lambda i: (0, i, 0)),
            pl.BlockSpec((SD, SD), lambda i: (0, 0)),
            pl.BlockSpec((E, E + SD), lambda i: (0, 0)),
            pl.BlockSpec((1, SD), lambda i: (0, 0)),
            pl.BlockSpec((1, E), lambda i: (0, 0)),
        ],
        out_specs=pl.BlockSpec((1, 1), lambda i: (0, 0),
                               memory_space=pltpu.SMEM),
        out_shape=jax.ShapeDtypeStruct((1, 1), jnp.float32),
    )(vI, WOy, SEy, WOn, SEn, f1w, f2w, f1b, f2b)


def kernel(x, y, neg, WI, WO, seq_embed, fc1_w, fc1_b, fc2_w, fc2_b):
    xi = x.astype(jnp.int32)
    yi = y.astype(jnp.int32)
    negt = neg.astype(jnp.int32).T.reshape(NEG * B)
    vI, WOy, SEy, WOn, SEn = _build_sc_gather()(xi, yi, negt, WI, WO,
                                                seq_embed)
    out = _tc_compute(vI, WOy, SEy,
                      WOn.reshape(NEG, B, E), SEn.reshape(NEG, B, SD),
                      fc1_w, fc2_w,
                      fc1_b.reshape(1, SD), fc2_b.reshape(1, E))
    return out[0, 0]


# speedup vs baseline: 4.9022x; 1.0444x over previous
import functools

import jax
import jax.numpy as jnp
from jax import lax
from jax.experimental import pallas as pl
from jax.experimental.pallas import tpu as pltpu
from jax.experimental.pallas import tpu_sc as plsc

B = 4096
E = 128
SD = 128
NEG = 5

NC = 2
NS = 16
NW = NC * NS
BPW = B // NW


@functools.cache
def _build_sc_gather():
    mesh = plsc.VectorSubcoreMesh(core_axis_name="c", subcore_axis_name="s")

    @functools.partial(
        pl.kernel,
        mesh=mesh,
        out_type=(
            jax.ShapeDtypeStruct((B, E), jnp.float32),
            jax.ShapeDtypeStruct((B, E), jnp.float32),
            jax.ShapeDtypeStruct((B, SD), jnp.float32),
            jax.ShapeDtypeStruct((NEG * B, E), jnp.float32),
            jax.ShapeDtypeStruct((NEG * B, SD), jnp.float32),
        ),
        scratch_types=[
            pltpu.VMEM((7, BPW), jnp.int32),
            pltpu.VMEM((BPW, E), jnp.float32),
            pltpu.VMEM((BPW, E), jnp.float32),
            pltpu.SemaphoreType.DMA,
            pltpu.SemaphoreType.DMA,
        ],
    )
    def _sc_gather(idx_all_h, WI_h, WO_h, SE_h,
                   vI_h, WOy_h, SEy_h, WOn_h, SEn_h,
                   idx_v, buf0, buf1, sem0, sem1):
        wid = lax.axis_index("s") * NC + lax.axis_index("c")
        base = wid * BPW

        pltpu.sync_copy(idx_all_h.at[wid], idx_v)

        tasks = [(0, WI_h, vI_h, base), (1, WO_h, WOy_h, base),
                 (1, SE_h, SEy_h, base)]
        for n in range(NEG):
            tasks.append((2 + n, WO_h, WOn_h, n * B + base))
        for n in range(NEG):
            tasks.append((2 + n, SE_h, SEn_h, n * B + base))

        bufs = (buf0, buf1)
        sems = (sem0, sem1)
        copies = [None, None]

        def start(t, slot):
            j, tab_h, _, _ = tasks[t]
            copies[slot] = pltpu.async_copy(tab_h.at[idx_v.at[j]],
                                            bufs[slot], sems[slot])

        start(0, 0)
        for t in range(len(tasks)):
            slot = t % 2
            if t + 1 < len(tasks):
                start(t + 1, 1 - slot)
            copies[slot].wait()
            _, _, out_h, ooff = tasks[t]
            pltpu.sync_copy(bufs[slot], out_h.at[pl.ds(ooff, BPW)])

    return _sc_gather


BC = 512


def _log_sigmoid(z):
    return jnp.minimum(z, 0.0) - jnp.log1p(jnp.exp(-jnp.abs(z)))


def _tc_body(vI_r, WOy_r, SEy_r, WOn_r, SEn_r, f1w_r, f2w_r, f1b_r, f2b_r,
             out_r):
    i = pl.program_id(0)
    vI = vI_r[...]
    U = jnp.dot(vI, f2w_r[...], preferred_element_type=jnp.float32)
    U1 = U[:, :E]
    U2 = U[:, E:]
    f1w = f1w_r[...]
    A1 = jnp.dot(U1, f1w, preferred_element_type=jnp.float32)
    A2 = jnp.dot(U2, f1w, preferred_element_type=jnp.float32)
    f1b = f1b_r[...]
    f2b = f2b_r[...]
    cI = jnp.sum(vI * f2b, axis=1)
    c1 = jnp.sum(U1 * f1b, axis=1)
    c2 = jnp.sum(U2 * f1b, axis=1)
    pos = jnp.sum(U1 * WOy_r[...] + A2 * SEy_r[...], axis=1) + c2 + cI
    part = -jnp.sum(_log_sigmoid(pos)) / B
    cneg = c1 + cI
    for n in range(NEG):
        zn = jnp.sum(A1 * WOn_r[n] + U2 * SEn_r[n], axis=1) + cneg
        part = part - jnp.sum(_log_sigmoid(-zn))

    @pl.when(i == 0)
    def _init():
        out_r[0, 0] = part

    @pl.when(i > 0)
    def _acc():
        out_r[0, 0] = out_r[0, 0] + part


def _tc_compute(vI, WOy, SEy, WOn, SEn, f1w, f2w, f1b, f2b):
    grid = (B // BC,)
    return pl.pallas_call(
        _tc_body,
        grid=grid,
        in_specs=[
            pl.BlockSpec((BC, E), lambda i: (i, 0)),
            pl.BlockSpec((BC, E), lambda i: (i, 0)),
            pl.BlockSpec((BC, SD), lambda i: (i, 0)),
            pl.BlockSpec((NEG, BC, E), lambda i: (0, i, 0)),
            pl.BlockSpec((NEG, BC, SD), lambda i: (0, i, 0)),
            pl.BlockSpec((SD, SD), lambda i: (0, 0)),
            pl.BlockSpec((E, E + SD), lambda i: (0, 0)),
            pl.BlockSpec((1, SD), lambda i: (0, 0)),
            pl.BlockSpec((1, E), lambda i: (0, 0)),
        ],
        out_specs=pl.BlockSpec((1, 1), lambda i: (0, 0),
                               memory_space=pltpu.SMEM),
        out_shape=jax.ShapeDtypeStruct((1, 1), jnp.float32),
    )(vI, WOy, SEy, WOn, SEn, f1w, f2w, f1b, f2b)


def kernel(x, y, neg, WI, WO, seq_embed, fc1_w, fc1_b, fc2_w, fc2_b):
    xi = x.astype(jnp.int32).reshape(NW, 1, BPW)
    yi = y.astype(jnp.int32).reshape(NW, 1, BPW)
    negr = neg.astype(jnp.int32).reshape(NW, BPW, NEG).transpose(0, 2, 1)
    idx_all = jnp.concatenate([xi, yi, negr], axis=1)
    vI, WOy, SEy, WOn, SEn = _build_sc_gather()(idx_all, WI, WO,
                                                seq_embed)
    out = _tc_compute(vI, WOy, SEy,
                      WOn.reshape(NEG, B, E), SEn.reshape(NEG, B, SD),
                      fc1_w, fc2_w,
                      fc1_b.reshape(1, SD), fc2_b.reshape(1, E))
    return out[0, 0]
